# baseline (speedup 1.0000x reference)
import functools

import jax
import jax.numpy as jnp
from jax.experimental import pallas as pl
from jax.experimental.pallas import tpu as pltpu
from jax.experimental.pallas import tpu_sc as plsc

_R_SC = 768
_SC_BR = 4
_TC_BLK = 256
_LANES = 16


def _quantize_block(lv_ref, x_ref, o_ref):
    l0, l1, l2 = lv_ref[0], lv_ref[1], lv_ref[2]
    t0 = (l0 + l1) * 0.5
    t1 = (l1 + l2) * 0.5
    x = x_ref[...]
    o_ref[...] = jnp.where(x > t1, l2, jnp.where(x > t0, l1, l0))


def _decode_block(lv_ref, c_ref, _alias_ref, o_ref):
    l0, l1, l2 = lv_ref[0], lv_ref[1], lv_ref[2]
    c = c_ref[...]
    q3 = jnp.floor(c * 0.015625)
    r3 = c - 64.0 * q3
    q2 = jnp.floor(r3 * 0.0625)
    r2 = r3 - 16.0 * q2
    q1 = jnp.floor(r2 * 0.25)
    q0 = r2 - 4.0 * q1

    def val(q):
        return jnp.where(q > 1.5, l2, jnp.where(q > 0.5, l1, l0))

    n = c.shape[1]
    o_ref[:, 0:n] = val(q0)
    o_ref[:, n:2 * n] = val(q1)
    o_ref[:, 2 * n:3 * n] = val(q2)
    o_ref[:, 3 * n:4 * n] = val(q3)


def _sc_codes(x2, consts, r_sc, d):
    n = d // 4
    mesh = plsc.VectorSubcoreMesh(core_axis_name="core",
                                  subcore_axis_name="subcore")

    @functools.partial(
        pl.kernel,
        out_type=jax.ShapeDtypeStruct((r_sc, n), jnp.float32),
        mesh=mesh,
        scratch_types=[pltpu.VMEM((2 * _LANES,), jnp.float32)],
    )
    def k(x_hbm, c_hbm, o_hbm, c_v):
        pltpu.sync_copy(c_hbm, c_v)

        def body(in_v, out_v):
            t0v = c_v[pl.ds(0, _LANES)]
            t1v = c_v[pl.ds(_LANES, _LANES)]
            zero = jnp.full((_LANES,), 0.0, jnp.float32)
            w1 = jnp.full((_LANES,), 1.0, jnp.float32)
            w4 = jnp.full((_LANES,), 4.0, jnp.float32)
            w16 = jnp.full((_LANES,), 16.0, jnp.float32)
            w64 = jnp.full((_LANES,), 64.0, jnp.float32)

            @pl.loop(0, _SC_BR)
            def _row(r):
                @pl.loop(0, n, step=_LANES)
                def _col(j):
                    x0 = in_v[r, pl.ds(j, _LANES)]
                    x1 = in_v[r, pl.ds(j + n, _LANES)]
                    x2v = in_v[r, pl.ds(j + 2 * n, _LANES)]
                    x3 = in_v[r, pl.ds(j + 3 * n, _LANES)]
                    code = jnp.where(x0 > t0v, w1, zero)
                    code = code + jnp.where(x0 > t1v, w1, zero)
                    code = code + jnp.where(x1 > t0v, w4, zero)
                    code = code + jnp.where(x1 > t1v, w4, zero)
                    code = code + jnp.where(x2v > t0v, w16, zero)
                    code = code + jnp.where(x2v > t1v, w16, zero)
                    code = code + jnp.where(x3 > t0v, w64, zero)
                    code = code + jnp.where(x3 > t1v, w64, zero)
                    out_v[r, pl.ds(j, _LANES)] = code

        pltpu.emit_pipeline(
            body,
            grid=(r_sc // _SC_BR,),
            in_specs=[pl.BlockSpec((_SC_BR, d), lambda i: (i, 0))],
            out_specs=[pl.BlockSpec((_SC_BR, n), lambda i: (i, 0))],
            core_axis_name=("core", "subcore"),
            dimension_semantics=(pltpu.PARALLEL,),
        )(x_hbm, o_hbm)

    return k(x2, consts)


def kernel(x, levels):
    b, c, d = x.shape
    rows = b * c
    x2 = x.reshape(rows, d)

    t0 = (levels[0] + levels[1]) * 0.5
    t1 = (levels[1] + levels[2]) * 0.5
    consts = jnp.concatenate([
        jnp.broadcast_to(t0, (_LANES,)),
        jnp.broadcast_to(t1, (_LANES,)),
    ]).astype(jnp.float32)

    codes = _sc_codes(x2, consts, _R_SC, d)

    off = _R_SC // _TC_BLK
    a_out = pl.pallas_call(
        _quantize_block,
        grid=((rows - _R_SC) // _TC_BLK,),
        in_specs=[
            pl.BlockSpec(memory_space=pltpu.MemorySpace.SMEM),
            pl.BlockSpec((_TC_BLK, d), lambda i: (i + off, 0)),
        ],
        out_specs=pl.BlockSpec((_TC_BLK, d), lambda i: (i + off, 0)),
        out_shape=jax.ShapeDtypeStruct((rows, d), x.dtype),
    )(levels, x2)

    out = pl.pallas_call(
        _decode_block,
        grid=(_R_SC // _TC_BLK,),
        in_specs=[
            pl.BlockSpec(memory_space=pltpu.MemorySpace.SMEM),
            pl.BlockSpec((_TC_BLK, d // 4), lambda i: (i, 0)),
            pl.BlockSpec(memory_space=pltpu.MemorySpace.HBM),
        ],
        out_specs=pl.BlockSpec((_TC_BLK, d), lambda i: (i, 0)),
        out_shape=jax.ShapeDtypeStruct((rows, d), x.dtype),
        input_output_aliases={2: 0},
    )(levels, codes, a_out)

    return out.reshape(b, c, d)


# speedup vs baseline: 1.3200x; 1.3200x over previous
import jax
import jax.numpy as jnp
from jax.experimental import pallas as pl
from jax.experimental.pallas import tpu as pltpu


def _quantize_block(lv_ref, x_ref, o_ref):
    l0, l1, l2 = lv_ref[0], lv_ref[1], lv_ref[2]
    t0 = (l0 + l1) * 0.5
    t1 = (l1 + l2) * 0.5
    x = x_ref[...]
    o_ref[...] = jnp.where(x > t1, l2, jnp.where(x > t0, l1, l0))


def kernel(x, levels):
    b, c, d = x.shape
    rows = b * c
    x2 = x.reshape(rows, d)
    block_rows = 256
    out = pl.pallas_call(
        _quantize_block,
        grid=(rows // block_rows,),
        in_specs=[
            pl.BlockSpec(memory_space=pltpu.MemorySpace.SMEM),
            pl.BlockSpec((block_rows, d), lambda i: (i, 0)),
        ],
        out_specs=pl.BlockSpec((block_rows, d), lambda i: (i, 0)),
        out_shape=jax.ShapeDtypeStruct((rows, d), x.dtype),
        compiler_params=pltpu.CompilerParams(
            dimension_semantics=("parallel",),
        ),
    )(levels, x2)
    return out.reshape(b, c, d)
